# baseline (speedup 1.0000x reference)
import jax
import jax.numpy as jnp
from jax.experimental import pallas as pl
from jax.experimental.pallas import tpu as pltpu

_CK = 256
_CKD = 2048
_BLK = 512


def _make_deg_kernel(nblk, ckd):
    def _deg_kernel(starts_ref, src_ref, s_ref):
        t = pl.program_id(0)
        tm_ = s_ref.shape[0]
        rows = t * tm_ + jax.lax.broadcasted_iota(jnp.int32, (tm_, 1), 0)
        start = starts_ref[t * nblk]
        end = starts_ref[(t + 1) * nblk]
        base = (start // ckd) * ckd
        nch = (end - base + ckd - 1) // ckd

        def body(k, cnt):
            off = pl.multiple_of(base + k * ckd, ckd)
            sl = src_ref[:, pl.ds(off, ckd)]
            m = (rows == sl).astype(jnp.float32)
            return cnt + jnp.sum(m, axis=1, keepdims=True)

        cnt = jax.lax.fori_loop(0, nch, body, jnp.ones((tm_, 1), jnp.float32))
        s_ref[...] = jax.lax.rsqrt(cnt)

    return _deg_kernel


def _xw1_kernel(x_ref, w1_ref, s_ref, o_ref):
    xb = x_ref[...].astype(jnp.bfloat16)
    z = jnp.dot(xb, w1_ref[...], preferred_element_type=jnp.float32)
    o_ref[...] = (z * s_ref[...]).astype(jnp.bfloat16)


def _sparse_agg(t, tm, n_pad, nblk, starts_ref, src_ref, dst_ref, m_ref,
                z_ref, rows):
    z_ref[...] = m_ref[pl.ds(t * tm, tm), :].astype(jnp.float32)
    for b in range(nblk):
        w_b = min(_BLK, n_pad - b * _BLK)
        start = starts_ref[t * nblk + b]
        end = starts_ref[t * nblk + b + 1]
        base = (start // _CK) * _CK
        nch = (end - base + _CK - 1) // _CK
        cols_b = b * _BLK + jax.lax.broadcasted_iota(jnp.int32, (w_b, 1), 0)
        slab = m_ref[b * _BLK:b * _BLK + w_b, :]

        def body(k, carry, cols_b=cols_b, slab=slab, base=base):
            off = pl.multiple_of(base + k * _CK, _CK)
            sl_src = src_ref[:, pl.ds(off, _CK)]
            sl_dst = dst_ref[:, pl.ds(off, _CK)]
            oh_src = (rows == sl_src).astype(jnp.bfloat16)
            oh_dst_t = (cols_b == sl_dst).astype(jnp.bfloat16)
            gath = jax.lax.dot_general(
                oh_dst_t, slab,
                dimension_numbers=(((0,), (0,)), ((), ())),
                preferred_element_type=jnp.float32)
            z_ref[...] += jax.lax.dot_general(
                oh_src, gath.astype(jnp.bfloat16),
                dimension_numbers=(((1,), (0,)), ((), ())),
                preferred_element_type=jnp.float32)
            return carry

        jax.lax.fori_loop(0, nch, body, 0)


def _make_agg1_kernel(tm, n_pad, nblk):
    def _agg1_kernel(starts_ref, src_ref, dst_ref, xw1_ref, b1_ref, w2_ref,
                     s_ref, g_ref, z1_ref):
        t = pl.program_id(0)
        rows = t * tm + jax.lax.broadcasted_iota(jnp.int32, (tm, 1), 0)
        _sparse_agg(t, tm, n_pad, nblk, starts_ref, src_ref, dst_ref,
                    xw1_ref, z1_ref, rows)
        st = s_ref[...]
        h1 = jnp.maximum(z1_ref[...] * st + b1_ref[...], 0.0
                         ).astype(jnp.bfloat16)
        g = jnp.dot(h1, w2_ref[...], preferred_element_type=jnp.float32)
        g_ref[...] = (g * st).astype(jnp.bfloat16)

    return _agg1_kernel


def _make_agg2_kernel(tm, n_pad, nblk):
    def _agg2_kernel(starts_ref, src_ref, dst_ref, g_in_ref, b2_ref, s_ref,
                     o_ref, z2_ref):
        t = pl.program_id(0)
        rows = t * tm + jax.lax.broadcasted_iota(jnp.int32, (tm, 1), 0)
        _sparse_agg(t, tm, n_pad, nblk, starts_ref, src_ref, dst_ref,
                    g_in_ref, z2_ref, rows)
        z = z2_ref[...] * s_ref[...] + b2_ref[...]
        m = jnp.max(z, axis=1, keepdims=True)
        zs = z - m
        lse = jnp.log(jnp.sum(jnp.exp(zs), axis=1, keepdims=True))
        o_ref[...] = zs - lse

    return _agg2_kernel


def _round_up(v, m):
    return ((v + m - 1) // m) * m


def kernel(x, edge_index, w1, b1, w2, b2):
    n, f_in = x.shape
    h = w1.shape[1]
    c = w2.shape[1]

    tm = 256 if n >= 512 else 128
    n_pad = _round_up(n, tm)
    nt = n_pad // tm
    nblk = -(-n_pad // _BLK)
    ngroups = nt * nblk

    src = edge_index[0].astype(jnp.int32)
    dst = edge_index[1].astype(jnp.int32)
    e = src.shape[0]
    e_pad = _round_up(e + 1, max(_CK, _CKD))
    sb = (n_pad - 1).bit_length()
    gkey = (src // tm) * nblk + dst // _BLK
    skey = (gkey << (2 * sb)) | (src << sb) | dst
    skey = jnp.concatenate(
        [skey, jnp.full((e_pad - e,), jnp.int32(2**31 - 1), jnp.int32)])
    skey_s = jnp.sort(skey)
    starts = jnp.searchsorted(
        skey_s,
        (jnp.arange(ngroups + 1, dtype=jnp.int32) << (2 * sb))
    ).astype(jnp.int32)
    src_s = (skey_s >> sb) & ((1 << sb) - 1)
    dst_s = skey_s & ((1 << sb) - 1)
    dup = jnp.concatenate(
        [jnp.zeros((1,), jnp.bool_), skey_s[1:] == skey_s[:-1]])
    invalid = dup | (src_s == dst_s) | (skey_s == 2**31 - 1)
    src_m = jnp.where(invalid, jnp.int32(1 << 20), src_s)
    src_p = src_m.reshape(1, e_pad)
    dst_p = dst_s.reshape(1, e_pad)

    if n_pad != n:
        x = jnp.zeros((n_pad, f_in), x.dtype).at[:n].set(x)

    w1_bf = w1.astype(jnp.bfloat16)
    w2_bf = w2.astype(jnp.bfloat16)
    b1_f = b1.astype(jnp.float32)
    b2_f = b2.astype(jnp.float32)

    grid = (nt,)
    cparams = pltpu.CompilerParams(
        dimension_semantics=("parallel",), vmem_limit_bytes=64 * 2**20)

    s2 = pl.pallas_call(
        _make_deg_kernel(nblk, _CKD),
        grid_spec=pltpu.PrefetchScalarGridSpec(
            num_scalar_prefetch=1,
            grid=grid,
            in_specs=[pl.BlockSpec((1, e_pad), lambda i, st: (0, 0))],
            out_specs=pl.BlockSpec((tm, 1), lambda i, st: (i, 0)),
        ),
        out_shape=jax.ShapeDtypeStruct((n_pad, 1), jnp.float32),
        compiler_params=cparams,
    )(starts, src_p)

    xw1p = pl.pallas_call(
        _xw1_kernel,
        out_shape=jax.ShapeDtypeStruct((n_pad, h), jnp.bfloat16),
        grid=grid,
        in_specs=[
            pl.BlockSpec((tm, f_in), lambda i: (i, 0)),
            pl.BlockSpec((f_in, h), lambda i: (0, 0)),
            pl.BlockSpec((tm, 1), lambda i: (i, 0)),
        ],
        out_specs=pl.BlockSpec((tm, h), lambda i: (i, 0)),
        compiler_params=cparams,
    )(x, w1_bf, s2)

    g = pl.pallas_call(
        _make_agg1_kernel(tm, n_pad, nblk),
        grid_spec=pltpu.PrefetchScalarGridSpec(
            num_scalar_prefetch=1,
            grid=grid,
            in_specs=[
                pl.BlockSpec((1, e_pad), lambda i, st: (0, 0)),
                pl.BlockSpec((1, e_pad), lambda i, st: (0, 0)),
                pl.BlockSpec((n_pad, h), lambda i, st: (0, 0)),
                pl.BlockSpec((1, h), lambda i, st: (0, 0)),
                pl.BlockSpec((h, c), lambda i, st: (0, 0)),
                pl.BlockSpec((tm, 1), lambda i, st: (i, 0)),
            ],
            out_specs=pl.BlockSpec((tm, c), lambda i, st: (i, 0)),
            scratch_shapes=[pltpu.VMEM((tm, h), jnp.float32)],
        ),
        out_shape=jax.ShapeDtypeStruct((n_pad, c), jnp.bfloat16),
        compiler_params=cparams,
    )(starts, src_p, dst_p, xw1p, b1_f, w2_bf, s2)

    out = pl.pallas_call(
        _make_agg2_kernel(tm, n_pad, nblk),
        grid_spec=pltpu.PrefetchScalarGridSpec(
            num_scalar_prefetch=1,
            grid=grid,
            in_specs=[
                pl.BlockSpec((1, e_pad), lambda i, st: (0, 0)),
                pl.BlockSpec((1, e_pad), lambda i, st: (0, 0)),
                pl.BlockSpec((n_pad, c), lambda i, st: (0, 0)),
                pl.BlockSpec((1, c), lambda i, st: (0, 0)),
                pl.BlockSpec((tm, 1), lambda i, st: (i, 0)),
            ],
            out_specs=pl.BlockSpec((tm, c), lambda i, st: (i, 0)),
            scratch_shapes=[pltpu.VMEM((tm, c), jnp.float32)],
        ),
        out_shape=jax.ShapeDtypeStruct((n_pad, c), jnp.float32),
        compiler_params=cparams,
    )(starts, src_p, dst_p, g, b2_f, s2)

    return out[:n]


# speedup vs baseline: 1.6470x; 1.6470x over previous
import jax
import jax.numpy as jnp
from jax.experimental import pallas as pl
from jax.experimental.pallas import tpu as pltpu

_CK = 256
_CKD = 2048
_BLK = 512


def _make_deg_kernel(nblk, ckd):
    def _deg_kernel(starts_ref, src_ref, s_ref):
        t = pl.program_id(0)
        tm_ = s_ref.shape[0]
        rows = t * tm_ + jax.lax.broadcasted_iota(jnp.int32, (tm_, 1), 0)
        start = starts_ref[t * nblk]
        end = starts_ref[(t + 1) * nblk]
        base = (start // ckd) * ckd
        nch = (end - base + ckd - 1) // ckd

        def body(k, cnt):
            off = pl.multiple_of(base + k * ckd, ckd)
            sl = src_ref[:, pl.ds(off, ckd)]
            m = (rows == sl).astype(jnp.float32)
            return cnt + jnp.sum(m, axis=1, keepdims=True)

        cnt = jax.lax.fori_loop(0, nch, body, jnp.ones((tm_, 1), jnp.float32))
        s_ref[...] = jax.lax.rsqrt(cnt)

    return _deg_kernel


def _xw1_kernel(x_ref, w1_ref, s_ref, o_ref):
    xb = x_ref[...].astype(jnp.bfloat16)
    z = jnp.dot(xb, w1_ref[...], preferred_element_type=jnp.float32)
    o_ref[...] = (z * s_ref[...]).astype(jnp.bfloat16)


def _make_agg1_kernel(tm, n_pad, nblk):
    def _agg1_kernel(starts_ref, src_ref, dst_ref, xw1_ref, b1_ref, w2_ref,
                     s_ref, adj_ref, g_ref, acc_ref, z1_ref):
        t = pl.program_id(0)
        rows = t * tm + jax.lax.broadcasted_iota(jnp.int32, (tm, 1), 0)
        z1_ref[...] = jnp.zeros_like(z1_ref)
        for b in range(nblk):
            w_b = min(_BLK, n_pad - b * _BLK)
            start = starts_ref[t * nblk + b]
            end = starts_ref[t * nblk + b + 1]
            base = (start // _CK) * _CK
            nch = (end - base + _CK - 1) // _CK
            cols_b = b * _BLK + jax.lax.broadcasted_iota(
                jnp.int32, (w_b, 1), 0)

            def one_chunk(off, cols_b=cols_b, w_b=w_b):
                off = pl.multiple_of(off, _CK)
                sl_src = src_ref[:, pl.ds(off, _CK)]
                sl_dst = dst_ref[:, pl.ds(off, _CK)]
                oh_src = (rows == sl_src).astype(jnp.bfloat16)
                oh_dst_t = (cols_b == sl_dst).astype(jnp.bfloat16)
                return jax.lax.dot_general(
                    oh_src, oh_dst_t,
                    dimension_numbers=(((1,), (1,)), ((), ())),
                    preferred_element_type=jnp.float32)

            @pl.when(nch > 1)
            def _(base=base, nch=nch, one_chunk=one_chunk, w_b=w_b):
                acc_ref[...] = jnp.zeros_like(acc_ref)

                def body(k, carry):
                    acc_ref[:, :w_b] += one_chunk(base + k * _CK)
                    return carry

                jax.lax.fori_loop(1, nch, body, 0)

            r = one_chunk(base)
            r = r + jnp.where(nch > 1, acc_ref[:, :w_b], 0.0)
            eye_b = (rows == cols_b.reshape(1, w_b)).astype(jnp.float32)
            blk = jnp.maximum(jnp.minimum(r, 1.0), eye_b).astype(jnp.bfloat16)
            adj_ref[:, b * _BLK:b * _BLK + w_b] = blk
            z1_ref[...] += jnp.dot(
                blk, xw1_ref[b * _BLK:b * _BLK + w_b, :],
                preferred_element_type=jnp.float32)
        st = s_ref[...]
        h1 = jnp.maximum(z1_ref[...] * st + b1_ref[...], 0.0
                         ).astype(jnp.bfloat16)
        g = jnp.dot(h1, w2_ref[...], preferred_element_type=jnp.float32)
        g_ref[...] = (g * st).astype(jnp.bfloat16)

    return _agg1_kernel


def _agg2_kernel(a_ref, g_ref, b2_ref, s_ref, o_ref):
    z = jnp.dot(a_ref[...], g_ref[...], preferred_element_type=jnp.float32)
    z = z * s_ref[...] + b2_ref[...]
    m = jnp.max(z, axis=1, keepdims=True)
    zs = z - m
    lse = jnp.log(jnp.sum(jnp.exp(zs), axis=1, keepdims=True))
    o_ref[...] = zs - lse


def _round_up(v, m):
    return ((v + m - 1) // m) * m


def kernel(x, edge_index, w1, b1, w2, b2):
    n, f_in = x.shape
    h = w1.shape[1]
    c = w2.shape[1]

    tm = 256 if n >= 512 else 128
    n_pad = _round_up(n, tm)
    nt = n_pad // tm
    nblk = -(-n_pad // _BLK)
    ngroups = nt * nblk

    src = edge_index[0].astype(jnp.int32)
    dst = edge_index[1].astype(jnp.int32)
    e = src.shape[0]
    e_pad = _round_up(e + 1, max(_CK, _CKD))
    sb = (n_pad - 1).bit_length()
    gkey = (src // tm) * nblk + dst // _BLK
    skey = (gkey << (2 * sb)) | (src << sb) | dst
    skey = jnp.concatenate(
        [skey, jnp.full((e_pad - e,), jnp.int32(2**31 - 1), jnp.int32)])
    skey_s = jnp.sort(skey)
    gk_sorted = skey_s >> (2 * sb)
    counts = jnp.sum(
        gk_sorted[None, :] == jnp.arange(ngroups, dtype=jnp.int32)[:, None],
        axis=1, dtype=jnp.int32)
    starts = jnp.concatenate(
        [jnp.zeros((1,), jnp.int32), jnp.cumsum(counts, dtype=jnp.int32)])
    src_s = (skey_s >> sb) & ((1 << sb) - 1)
    dst_s = skey_s & ((1 << sb) - 1)
    dup = jnp.concatenate(
        [jnp.zeros((1,), jnp.bool_), skey_s[1:] == skey_s[:-1]])
    invalid = dup | (src_s == dst_s) | (skey_s == 2**31 - 1)
    src_m = jnp.where(invalid, jnp.int32(1 << 20), src_s)
    src_p = src_m.reshape(1, e_pad)
    dst_p = dst_s.reshape(1, e_pad)

    if n_pad != n:
        x = jnp.zeros((n_pad, f_in), x.dtype).at[:n].set(x)

    w1_bf = w1.astype(jnp.bfloat16)
    w2_bf = w2.astype(jnp.bfloat16)
    b1_f = b1.astype(jnp.float32)
    b2_f = b2.astype(jnp.float32)

    grid = (nt,)
    cparams = pltpu.CompilerParams(
        dimension_semantics=("parallel",), vmem_limit_bytes=64 * 2**20)

    s2 = pl.pallas_call(
        _make_deg_kernel(nblk, _CKD),
        grid_spec=pltpu.PrefetchScalarGridSpec(
            num_scalar_prefetch=1,
            grid=grid,
            in_specs=[pl.BlockSpec((1, e_pad), lambda i, st: (0, 0))],
            out_specs=pl.BlockSpec((tm, 1), lambda i, st: (i, 0)),
        ),
        out_shape=jax.ShapeDtypeStruct((n_pad, 1), jnp.float32),
        compiler_params=cparams,
    )(starts, src_p)

    xw1p = pl.pallas_call(
        _xw1_kernel,
        out_shape=jax.ShapeDtypeStruct((n_pad, h), jnp.bfloat16),
        grid=grid,
        in_specs=[
            pl.BlockSpec((tm, f_in), lambda i: (i, 0)),
            pl.BlockSpec((f_in, h), lambda i: (0, 0)),
            pl.BlockSpec((tm, 1), lambda i: (i, 0)),
        ],
        out_specs=pl.BlockSpec((tm, h), lambda i: (i, 0)),
        compiler_params=cparams,
    )(x, w1_bf, s2)

    adj, g = pl.pallas_call(
        _make_agg1_kernel(tm, n_pad, nblk),
        grid_spec=pltpu.PrefetchScalarGridSpec(
            num_scalar_prefetch=1,
            grid=grid,
            in_specs=[
                pl.BlockSpec((1, e_pad), lambda i, st: (0, 0)),
                pl.BlockSpec((1, e_pad), lambda i, st: (0, 0)),
                pl.BlockSpec((n_pad, h), lambda i, st: (0, 0)),
                pl.BlockSpec((1, h), lambda i, st: (0, 0)),
                pl.BlockSpec((h, c), lambda i, st: (0, 0)),
                pl.BlockSpec((tm, 1), lambda i, st: (i, 0)),
            ],
            out_specs=[
                pl.BlockSpec((tm, n_pad), lambda i, st: (i, 0)),
                pl.BlockSpec((tm, c), lambda i, st: (i, 0)),
            ],
            scratch_shapes=[
                pltpu.VMEM((tm, _BLK), jnp.float32),
                pltpu.VMEM((tm, h), jnp.float32),
            ],
        ),
        out_shape=[
            jax.ShapeDtypeStruct((n_pad, n_pad), jnp.bfloat16),
            jax.ShapeDtypeStruct((n_pad, c), jnp.bfloat16),
        ],
        compiler_params=cparams,
    )(starts, src_p, dst_p, xw1p, b1_f, w2_bf, s2)

    out = pl.pallas_call(
        _agg2_kernel,
        out_shape=jax.ShapeDtypeStruct((n_pad, c), jnp.float32),
        grid=grid,
        in_specs=[
            pl.BlockSpec((tm, n_pad), lambda i: (i, 0)),
            pl.BlockSpec((n_pad, c), lambda i: (0, 0)),
            pl.BlockSpec((1, c), lambda i: (0, 0)),
            pl.BlockSpec((tm, 1), lambda i: (i, 0)),
        ],
        out_specs=pl.BlockSpec((tm, c), lambda i: (i, 0)),
        compiler_params=cparams,
    )(adj, g, b2_f, s2)

    return out[:n]


# speedup vs baseline: 1.6578x; 1.0065x over previous
import jax
import jax.numpy as jnp
from jax.experimental import pallas as pl
from jax.experimental.pallas import tpu as pltpu

_CK = 256
_CKD = 2048
_BLK = 640


def _make_deg_kernel(nblk, ckd):
    def _deg_kernel(starts_ref, src_ref, s_ref):
        t = pl.program_id(0)
        tm_ = s_ref.shape[0]
        rows = t * tm_ + jax.lax.broadcasted_iota(jnp.int32, (tm_, 1), 0)
        start = starts_ref[t * nblk]
        end = starts_ref[(t + 1) * nblk]
        base = (start // ckd) * ckd
        nch = (end - base + ckd - 1) // ckd

        def body(k, cnt):
            off = pl.multiple_of(base + k * ckd, ckd)
            sl = src_ref[:, pl.ds(off, ckd)]
            m = (rows == sl).astype(jnp.float32)
            return cnt + jnp.sum(m, axis=1, keepdims=True)

        cnt = jax.lax.fori_loop(0, nch, body, jnp.ones((tm_, 1), jnp.float32))
        s_ref[...] = jax.lax.rsqrt(cnt)

    return _deg_kernel


def _xw1_kernel(x_ref, w1_ref, s_ref, o_ref):
    xb = x_ref[...].astype(jnp.bfloat16)
    z = jnp.dot(xb, w1_ref[...], preferred_element_type=jnp.float32)
    o_ref[...] = (z * s_ref[...]).astype(jnp.bfloat16)


def _make_agg1_kernel(tm, n_pad, nblk):
    def _agg1_kernel(starts_ref, src_ref, dst_ref, xw1_ref, b1_ref, w2_ref,
                     s_ref, adj_ref, g_ref, acc_ref, z1_ref):
        t = pl.program_id(0)
        rows = t * tm + jax.lax.broadcasted_iota(jnp.int32, (tm, 1), 0)
        z1_ref[...] = jnp.zeros_like(z1_ref)
        for b in range(nblk):
            w_b = min(_BLK, n_pad - b * _BLK)
            start = starts_ref[t * nblk + b]
            end = starts_ref[t * nblk + b + 1]
            base = (start // _CK) * _CK
            nch = (end - base + _CK - 1) // _CK
            cols_b = b * _BLK + jax.lax.broadcasted_iota(
                jnp.int32, (w_b, 1), 0)

            def one_chunk(off, cols_b=cols_b, w_b=w_b):
                off = pl.multiple_of(off, _CK)
                sl_src = src_ref[:, pl.ds(off, _CK)]
                sl_dst = dst_ref[:, pl.ds(off, _CK)]
                oh_src = (rows == sl_src).astype(jnp.bfloat16)
                oh_dst_t = (cols_b == sl_dst).astype(jnp.bfloat16)
                return jax.lax.dot_general(
                    oh_src, oh_dst_t,
                    dimension_numbers=(((1,), (1,)), ((), ())),
                    preferred_element_type=jnp.float32)

            @pl.when(nch > 1)
            def _(base=base, nch=nch, one_chunk=one_chunk, w_b=w_b):
                acc_ref[...] = jnp.zeros_like(acc_ref)

                def body(k, carry):
                    acc_ref[:, :w_b] += one_chunk(base + k * _CK)
                    return carry

                jax.lax.fori_loop(1, nch, body, 0)

            r = one_chunk(base)
            r = r + jnp.where(nch > 1, acc_ref[:, :w_b], 0.0)
            eye_b = (rows == cols_b.reshape(1, w_b)).astype(jnp.float32)
            blk = jnp.maximum(jnp.minimum(r, 1.0), eye_b).astype(jnp.bfloat16)
            adj_ref[:, b * _BLK:b * _BLK + w_b] = blk
            z1_ref[...] += jnp.dot(
                blk, xw1_ref[b * _BLK:b * _BLK + w_b, :],
                preferred_element_type=jnp.float32)
        st = s_ref[...]
        h1 = jnp.maximum(z1_ref[...] * st + b1_ref[...], 0.0
                         ).astype(jnp.bfloat16)
        g = jnp.dot(h1, w2_ref[...], preferred_element_type=jnp.float32)
        g_ref[...] = (g * st).astype(jnp.bfloat16)

    return _agg1_kernel


def _agg2_kernel(a_ref, g_ref, b2_ref, s_ref, o_ref):
    z = jnp.dot(a_ref[...], g_ref[...], preferred_element_type=jnp.float32)
    z = z * s_ref[...] + b2_ref[...]
    m = jnp.max(z, axis=1, keepdims=True)
    zs = z - m
    lse = jnp.log(jnp.sum(jnp.exp(zs), axis=1, keepdims=True))
    o_ref[...] = zs - lse


def _round_up(v, m):
    return ((v + m - 1) // m) * m


def kernel(x, edge_index, w1, b1, w2, b2):
    n, f_in = x.shape
    h = w1.shape[1]
    c = w2.shape[1]

    tm = 256 if n >= 512 else 128
    n_pad = _round_up(n, tm)
    nt = n_pad // tm
    nblk = -(-n_pad // _BLK)
    ngroups = nt * nblk

    src = edge_index[0].astype(jnp.int32)
    dst = edge_index[1].astype(jnp.int32)
    e = src.shape[0]
    e_pad = _round_up(e + 1, max(_CK, _CKD))
    sb = (n_pad - 1).bit_length()
    gkey = (src // tm) * nblk + dst // _BLK
    skey = (gkey << (2 * sb)) | (src << sb) | dst
    skey = jnp.concatenate(
        [skey, jnp.full((e_pad - e,), jnp.int32(2**31 - 1), jnp.int32)])
    skey_s = jnp.sort(skey)
    gk_sorted = skey_s >> (2 * sb)
    counts = jnp.sum(
        gk_sorted[None, :] == jnp.arange(ngroups, dtype=jnp.int32)[:, None],
        axis=1, dtype=jnp.int32)
    starts = jnp.concatenate(
        [jnp.zeros((1,), jnp.int32), jnp.cumsum(counts, dtype=jnp.int32)])
    src_s = (skey_s >> sb) & ((1 << sb) - 1)
    dst_s = skey_s & ((1 << sb) - 1)
    dup = jnp.concatenate(
        [jnp.zeros((1,), jnp.bool_), skey_s[1:] == skey_s[:-1]])
    invalid = dup | (src_s == dst_s) | (skey_s == 2**31 - 1)
    src_m = jnp.where(invalid, jnp.int32(1 << 20), src_s)
    src_p = src_m.reshape(1, e_pad)
    dst_p = dst_s.reshape(1, e_pad)

    if n_pad != n:
        x = jnp.zeros((n_pad, f_in), x.dtype).at[:n].set(x)

    w1_bf = w1.astype(jnp.bfloat16)
    w2_bf = w2.astype(jnp.bfloat16)
    b1_f = b1.astype(jnp.float32)
    b2_f = b2.astype(jnp.float32)

    grid = (nt,)
    cparams = pltpu.CompilerParams(
        dimension_semantics=("parallel",), vmem_limit_bytes=64 * 2**20)

    s2 = pl.pallas_call(
        _make_deg_kernel(nblk, _CKD),
        grid_spec=pltpu.PrefetchScalarGridSpec(
            num_scalar_prefetch=1,
            grid=grid,
            in_specs=[pl.BlockSpec((1, e_pad), lambda i, st: (0, 0))],
            out_specs=pl.BlockSpec((tm, 1), lambda i, st: (i, 0)),
        ),
        out_shape=jax.ShapeDtypeStruct((n_pad, 1), jnp.float32),
        compiler_params=cparams,
    )(starts, src_p)

    xw1p = pl.pallas_call(
        _xw1_kernel,
        out_shape=jax.ShapeDtypeStruct((n_pad, h), jnp.bfloat16),
        grid=grid,
        in_specs=[
            pl.BlockSpec((tm, f_in), lambda i: (i, 0)),
            pl.BlockSpec((f_in, h), lambda i: (0, 0)),
            pl.BlockSpec((tm, 1), lambda i: (i, 0)),
        ],
        out_specs=pl.BlockSpec((tm, h), lambda i: (i, 0)),
        compiler_params=cparams,
    )(x, w1_bf, s2)

    adj, g = pl.pallas_call(
        _make_agg1_kernel(tm, n_pad, nblk),
        grid_spec=pltpu.PrefetchScalarGridSpec(
            num_scalar_prefetch=1,
            grid=grid,
            in_specs=[
                pl.BlockSpec((1, e_pad), lambda i, st: (0, 0)),
                pl.BlockSpec((1, e_pad), lambda i, st: (0, 0)),
                pl.BlockSpec((n_pad, h), lambda i, st: (0, 0)),
                pl.BlockSpec((1, h), lambda i, st: (0, 0)),
                pl.BlockSpec((h, c), lambda i, st: (0, 0)),
                pl.BlockSpec((tm, 1), lambda i, st: (i, 0)),
            ],
            out_specs=[
                pl.BlockSpec((tm, n_pad), lambda i, st: (i, 0)),
                pl.BlockSpec((tm, c), lambda i, st: (i, 0)),
            ],
            scratch_shapes=[
                pltpu.VMEM((tm, _BLK), jnp.float32),
                pltpu.VMEM((tm, h), jnp.float32),
            ],
        ),
        out_shape=[
            jax.ShapeDtypeStruct((n_pad, n_pad), jnp.bfloat16),
            jax.ShapeDtypeStruct((n_pad, c), jnp.bfloat16),
        ],
        compiler_params=cparams,
    )(starts, src_p, dst_p, xw1p, b1_f, w2_bf, s2)

    out = pl.pallas_call(
        _agg2_kernel,
        out_shape=jax.ShapeDtypeStruct((n_pad, c), jnp.float32),
        grid=grid,
        in_specs=[
            pl.BlockSpec((tm, n_pad), lambda i: (i, 0)),
            pl.BlockSpec((n_pad, c), lambda i: (0, 0)),
            pl.BlockSpec((1, c), lambda i: (0, 0)),
            pl.BlockSpec((tm, 1), lambda i: (i, 0)),
        ],
        out_specs=pl.BlockSpec((tm, c), lambda i: (i, 0)),
        compiler_params=cparams,
    )(adj, g, b2_f, s2)

    return out[:n]


# speedup vs baseline: 1.7365x; 1.0475x over previous
import jax
import jax.numpy as jnp
from jax.experimental import pallas as pl
from jax.experimental.pallas import tpu as pltpu

_CK = 256
_CKD = 2048
_BLK = 640


def _make_xw1_kernel(nblk, ckd):
    def _xw1_kernel(starts_ref, src_ref, x_ref, w1_ref, o_ref, s_ref):
        t = pl.program_id(0)
        tm_ = o_ref.shape[0]
        rows = t * tm_ + jax.lax.broadcasted_iota(jnp.int32, (tm_, 1), 0)
        start = starts_ref[t * nblk]
        end = starts_ref[(t + 1) * nblk]
        base = (start // ckd) * ckd
        nch = (end - base + ckd - 1) // ckd

        def body(k, cnt):
            off = pl.multiple_of(base + k * ckd, ckd)
            sl = src_ref[:, pl.ds(off, ckd)]
            m = (rows == sl).astype(jnp.float32)
            return cnt + jnp.sum(m, axis=1, keepdims=True)

        cnt = jax.lax.fori_loop(0, nch, body, jnp.ones((tm_, 1), jnp.float32))
        st = jax.lax.rsqrt(cnt)
        s_ref[...] = st
        xb = x_ref[...].astype(jnp.bfloat16)
        z = jnp.dot(xb, w1_ref[...], preferred_element_type=jnp.float32)
        o_ref[...] = (z * st).astype(jnp.bfloat16)

    return _xw1_kernel


def _make_agg1_kernel(tm, n_pad, nblk):
    def _agg1_kernel(starts_ref, src_ref, dst_ref, xw1_ref, b1_ref, w2_ref,
                     s_ref, adj_ref, g_ref, acc_ref, z1_ref):
        t = pl.program_id(0)
        rows = t * tm + jax.lax.broadcasted_iota(jnp.int32, (tm, 1), 0)
        z1_ref[...] = jnp.zeros_like(z1_ref)
        for b in range(nblk):
            w_b = min(_BLK, n_pad - b * _BLK)
            start = starts_ref[t * nblk + b]
            end = starts_ref[t * nblk + b + 1]
            base = (start // _CK) * _CK
            nch = (end - base + _CK - 1) // _CK
            cols_b = b * _BLK + jax.lax.broadcasted_iota(
                jnp.int32, (w_b, 1), 0)

            def one_chunk(off, cols_b=cols_b, w_b=w_b):
                off = pl.multiple_of(off, _CK)
                sl_src = src_ref[:, pl.ds(off, _CK)]
                sl_dst = dst_ref[:, pl.ds(off, _CK)]
                oh_src = (rows == sl_src).astype(jnp.bfloat16)
                oh_dst_t = (cols_b == sl_dst).astype(jnp.bfloat16)
                return jax.lax.dot_general(
                    oh_src, oh_dst_t,
                    dimension_numbers=(((1,), (1,)), ((), ())),
                    preferred_element_type=jnp.float32)

            @pl.when(nch > 1)
            def _(base=base, nch=nch, one_chunk=one_chunk, w_b=w_b):
                acc_ref[...] = jnp.zeros_like(acc_ref)

                def body(k, carry):
                    acc_ref[:, :w_b] += one_chunk(base + k * _CK)
                    return carry

                jax.lax.fori_loop(1, nch, body, 0)

            r = one_chunk(base)
            r = r + jnp.where(nch > 1, acc_ref[:, :w_b], 0.0)
            eye_b = (rows == cols_b.reshape(1, w_b)).astype(jnp.float32)
            blk = jnp.maximum(jnp.minimum(r, 1.0), eye_b).astype(jnp.bfloat16)
            adj_ref[:, b * _BLK:b * _BLK + w_b] = blk
            z1_ref[...] += jnp.dot(
                blk, xw1_ref[b * _BLK:b * _BLK + w_b, :],
                preferred_element_type=jnp.float32)
        st = s_ref[...]
        h1 = jnp.maximum(z1_ref[...] * st + b1_ref[...], 0.0
                         ).astype(jnp.bfloat16)
        g = jnp.dot(h1, w2_ref[...], preferred_element_type=jnp.float32)
        g_ref[...] = (g * st).astype(jnp.bfloat16)

    return _agg1_kernel


def _agg2_kernel(a_ref, g_ref, b2_ref, s_ref, o_ref):
    z = jnp.dot(a_ref[...], g_ref[...], preferred_element_type=jnp.float32)
    z = z * s_ref[...] + b2_ref[...]
    m = jnp.max(z, axis=1, keepdims=True)
    zs = z - m
    lse = jnp.log(jnp.sum(jnp.exp(zs), axis=1, keepdims=True))
    o_ref[...] = zs - lse


def _round_up(v, m):
    return ((v + m - 1) // m) * m


def kernel(x, edge_index, w1, b1, w2, b2):
    n, f_in = x.shape
    h = w1.shape[1]
    c = w2.shape[1]

    tm = 256 if n >= 512 else 128
    n_pad = _round_up(n, tm)
    nt = n_pad // tm
    nblk = -(-n_pad // _BLK)
    ngroups = nt * nblk

    src = edge_index[0].astype(jnp.int32)
    dst = edge_index[1].astype(jnp.int32)
    e = src.shape[0]
    e_pad = _round_up(e + 1, max(_CK, _CKD))
    sb = (n_pad - 1).bit_length()
    gkey = (src // tm) * nblk + dst // _BLK
    skey = (gkey << (2 * sb)) | (src << sb) | dst
    skey = jnp.concatenate(
        [skey, jnp.full((e_pad - e,), jnp.int32(2**31 - 1), jnp.int32)])
    skey_s = jnp.sort(skey)
    gk_sorted = skey_s >> (2 * sb)
    counts = jnp.sum(
        gk_sorted[None, :] == jnp.arange(ngroups, dtype=jnp.int32)[:, None],
        axis=1, dtype=jnp.int32)
    starts = jnp.concatenate(
        [jnp.zeros((1,), jnp.int32), jnp.cumsum(counts, dtype=jnp.int32)])
    src_s = (skey_s >> sb) & ((1 << sb) - 1)
    dst_s = skey_s & ((1 << sb) - 1)
    dup = jnp.concatenate(
        [jnp.zeros((1,), jnp.bool_), skey_s[1:] == skey_s[:-1]])
    invalid = dup | (src_s == dst_s) | (skey_s == 2**31 - 1)
    src_m = jnp.where(invalid, jnp.int32(1 << 20), src_s)
    src_p = src_m.reshape(1, e_pad)
    dst_p = dst_s.reshape(1, e_pad)

    if n_pad != n:
        x = jnp.zeros((n_pad, f_in), x.dtype).at[:n].set(x)

    w1_bf = w1.astype(jnp.bfloat16)
    w2_bf = w2.astype(jnp.bfloat16)
    b1_f = b1.astype(jnp.float32)
    b2_f = b2.astype(jnp.float32)

    grid = (nt,)
    cparams = pltpu.CompilerParams(
        dimension_semantics=("parallel",), vmem_limit_bytes=64 * 2**20)

    xw1p, s2 = pl.pallas_call(
        _make_xw1_kernel(nblk, _CKD),
        grid_spec=pltpu.PrefetchScalarGridSpec(
            num_scalar_prefetch=1,
            grid=grid,
            in_specs=[
                pl.BlockSpec((1, e_pad), lambda i, st: (0, 0)),
                pl.BlockSpec((tm, f_in), lambda i, st: (i, 0)),
                pl.BlockSpec((f_in, h), lambda i, st: (0, 0)),
            ],
            out_specs=[
                pl.BlockSpec((tm, h), lambda i, st: (i, 0)),
                pl.BlockSpec((tm, 1), lambda i, st: (i, 0)),
            ],
        ),
        out_shape=[
            jax.ShapeDtypeStruct((n_pad, h), jnp.bfloat16),
            jax.ShapeDtypeStruct((n_pad, 1), jnp.float32),
        ],
        compiler_params=cparams,
    )(starts, src_p, x, w1_bf)

    adj, g = pl.pallas_call(
        _make_agg1_kernel(tm, n_pad, nblk),
        grid_spec=pltpu.PrefetchScalarGridSpec(
            num_scalar_prefetch=1,
            grid=grid,
            in_specs=[
                pl.BlockSpec((1, e_pad), lambda i, st: (0, 0)),
                pl.BlockSpec((1, e_pad), lambda i, st: (0, 0)),
                pl.BlockSpec((n_pad, h), lambda i, st: (0, 0)),
                pl.BlockSpec((1, h), lambda i, st: (0, 0)),
                pl.BlockSpec((h, c), lambda i, st: (0, 0)),
                pl.BlockSpec((tm, 1), lambda i, st: (i, 0)),
            ],
            out_specs=[
                pl.BlockSpec((tm, n_pad), lambda i, st: (i, 0)),
                pl.BlockSpec((tm, c), lambda i, st: (i, 0)),
            ],
            scratch_shapes=[
                pltpu.VMEM((tm, _BLK), jnp.float32),
                pltpu.VMEM((tm, h), jnp.float32),
            ],
        ),
        out_shape=[
            jax.ShapeDtypeStruct((n_pad, n_pad), jnp.bfloat16),
            jax.ShapeDtypeStruct((n_pad, c), jnp.bfloat16),
        ],
        compiler_params=cparams,
    )(starts, src_p, dst_p, xw1p, b1_f, w2_bf, s2)

    out = pl.pallas_call(
        _agg2_kernel,
        out_shape=jax.ShapeDtypeStruct((n_pad, c), jnp.float32),
        grid=grid,
        in_specs=[
            pl.BlockSpec((tm, n_pad), lambda i: (i, 0)),
            pl.BlockSpec((n_pad, c), lambda i: (0, 0)),
            pl.BlockSpec((1, c), lambda i: (0, 0)),
            pl.BlockSpec((tm, 1), lambda i: (i, 0)),
        ],
        out_specs=pl.BlockSpec((tm, c), lambda i: (i, 0)),
        compiler_params=cparams,
    )(adj, g, b2_f, s2)

    return out[:n]
